# baseline (speedup 1.0000x reference)
import functools

import jax
import jax.numpy as jnp
import numpy as np
from jax import lax
from jax.experimental import pallas as pl
from jax.experimental.pallas import tpu as pltpu
from jax.experimental.pallas import tpu_sc as plsc

_DIN = 48
_DOUT = 192
_ROW = _DIN * 3 * _DIN
_OROW = _DOUT * _DOUT
_NW = 32
_PW = _DOUT // _NW


def _sc_body(a_hbm, o_hbm, t0, t1, db, hu, ob, sems):
    wid = lax.axis_index("s") * 2 + lax.axis_index("c")
    od0 = wid * _PW

    iota = lax.iota(jnp.int32, 16)
    iota3 = iota * 3
    w0idx = []
    w1idx = []
    wfr = []
    for ch in range(12):
        owv = iota + 16 * ch
        c8 = 2 * owv - 3
        iw0 = jnp.right_shift(c8, 3)
        fr = (c8 - 8 * iw0).astype(jnp.float32) * 0.125
        w0idx.append(jnp.clip(iw0, 0, _DIN - 1))
        w1idx.append(jnp.clip(iw0 + 1, 0, _DIN - 1))
        wfr.append(fr)

    handles = [None, None]
    for k in range(_PW):
        od = od0 + k
        c8 = 2 * od - 3
        i0 = jnp.right_shift(c8, 3)
        dfr = (c8 - 8 * i0).astype(jnp.float32) * 0.125
        i0c = jnp.clip(i0, 0, _DIN - 1)
        i1c = jnp.clip(i0 + 1, 0, _DIN - 1)
        pltpu.sync_copy(a_hbm.at[i0c], t0)
        pltpu.sync_copy(a_hbm.at[i1c], t1)

        def dblend(j, _):
            s = pl.ds(j * 16, 16)
            v0 = t0[s]
            db[s] = v0 + dfr * (t1[s] - v0)
            return 0

        lax.fori_loop(0, _ROW // 16, dblend, 0)

        for c in range(3):
            def hbody(oh, _, c=c):
                h8 = 2 * oh - 3
                ih0 = jnp.right_shift(h8, 3)
                hfr = (h8 - 8 * ih0).astype(jnp.float32) * 0.125
                ih0c = jnp.clip(ih0, 0, _DIN - 1)
                ih1c = jnp.clip(ih0 + 1, 0, _DIN - 1)
                b0 = jnp.full((16,), ih0c * 144, jnp.int32)
                b1 = jnp.full((16,), ih1c * 144, jnp.int32)
                for ch in range(3):
                    off = 48 * ch + c
                    g0 = plsc.load_gather(db, [b0 + off + iota3])
                    g1 = plsc.load_gather(db, [b1 + off + iota3])
                    hu[pl.ds(oh * _DIN + ch * 16, 16)] = g0 + hfr * (g1 - g0)
                return 0

            lax.fori_loop(0, _DOUT, hbody, 0)

            p = k * 3 + c
            slot = p % 2
            if handles[slot] is not None:
                handles[slot].wait()

            def wbody(oh, _, slot=slot):
                rb = jnp.full((16,), oh * _DIN, jnp.int32)
                for ch in range(12):
                    g0 = plsc.load_gather(hu, [rb + w0idx[ch]])
                    g1 = plsc.load_gather(hu, [rb + w1idx[ch]])
                    ob[slot, pl.ds(oh * _DOUT + ch * 16, 16)] = (
                        g0 + wfr[ch] * (g1 - g0)
                    )
                return 0

            lax.fori_loop(0, _DOUT, wbody, 0)
            row = (c * _DOUT + od0) + k
            handles[slot] = pltpu.async_copy(
                ob.at[slot], o_hbm.at[row], sems.at[slot]
            )
    for h in handles:
        if h is not None:
            h.wait()


@jax.jit
def _sc_upsample(disp):
    a = jnp.reshape(disp, (_DIN, _ROW))
    mesh = plsc.VectorSubcoreMesh(core_axis_name="c", subcore_axis_name="s")
    out = pl.kernel(
        _sc_body,
        mesh=mesh,
        compiler_params=pltpu.CompilerParams(
            needs_layout_passes=False, use_tc_tiling_on_sc=False
        ),
        out_type=jax.ShapeDtypeStruct((3 * _DOUT, _OROW), jnp.float32),
        scratch_types=[
            pltpu.VMEM((_ROW,), jnp.float32),
            pltpu.VMEM((_ROW,), jnp.float32),
            pltpu.VMEM((_ROW,), jnp.float32),
            pltpu.VMEM((_DOUT * _DIN,), jnp.float32),
            pltpu.VMEM((2, _OROW), jnp.float32),
            pltpu.SemaphoreType.DMA((2,)),
        ],
    )(a)
    return jnp.reshape(out, (1, 3, _DOUT, _DOUT, _DOUT))


def kernel(kpts, disp, features_fixed):
    del kpts, features_fixed
    return _sc_upsample(disp)


# speedup vs baseline: 2.1188x; 2.1188x over previous
import functools

import jax
import jax.numpy as jnp
import numpy as np
from jax import lax
from jax.experimental import pallas as pl
from jax.experimental.pallas import tpu as pltpu
from jax.experimental.pallas import tpu_sc as plsc

_DIN = 48
_DOUT = 192
_ROW = _DIN * 3 * _DIN
_OROW = _DOUT * _DOUT
_NW = 32
_PW = _DOUT // _NW


def _sc_body(a_hbm, o_hbm, t0, t1, db, hu, ob, sems):
    wid = lax.axis_index("s") * 2 + lax.axis_index("c")
    od0 = wid * _PW

    iota = lax.iota(jnp.int32, 16)
    iota3 = iota * 3
    w0idx = []
    w1idx = []
    wfr = []
    for ch in range(12):
        owv = iota + 16 * ch
        c8 = 2 * owv - 3
        iw0 = jnp.right_shift(c8, 3)
        fr = (c8 - 8 * iw0).astype(jnp.float32) * 0.125
        w0idx.append(jnp.clip(iw0, 0, _DIN - 1))
        w1idx.append(jnp.clip(iw0 + 1, 0, _DIN - 1))
        wfr.append(fr)

    handles = [None, None]
    for k in range(_PW):
        od = od0 + k
        c8 = 2 * od - 3
        i0 = jnp.right_shift(c8, 3)
        dfr = (c8 - 8 * i0).astype(jnp.float32) * 0.125
        i0c = jnp.clip(i0, 0, _DIN - 1)
        i1c = jnp.clip(i0 + 1, 0, _DIN - 1)
        pltpu.sync_copy(a_hbm.at[i0c], t0)
        pltpu.sync_copy(a_hbm.at[i1c], t1)

        @plsc.parallel_loop(0, _ROW // 16, unroll=4)
        def _(j):
            s = pl.ds(j * 16, 16)
            v0 = t0[s]
            db[s] = v0 + dfr * (t1[s] - v0)

        for c in range(3):
            def hbody(oh, c=c):
                h8 = 2 * oh - 3
                ih0 = jnp.right_shift(h8, 3)
                hfr = (h8 - 8 * ih0).astype(jnp.float32) * 0.125
                ih0c = jnp.clip(ih0, 0, _DIN - 1)
                ih1c = jnp.clip(ih0 + 1, 0, _DIN - 1)
                b0 = jnp.full((16,), ih0c * 144, jnp.int32)
                b1 = jnp.full((16,), ih1c * 144, jnp.int32)
                for ch in range(3):
                    off = 48 * ch + c
                    g0 = plsc.load_gather(db, [b0 + off + iota3])
                    g1 = plsc.load_gather(db, [b1 + off + iota3])
                    hu[pl.ds(oh * _DIN + ch * 16, 16)] = g0 + hfr * (g1 - g0)

            plsc.parallel_loop(0, _DOUT, unroll=1)(hbody)

            p = k * 3 + c
            slot = p % 2
            if handles[slot] is not None:
                handles[slot].wait()

            def wbody(oh, slot=slot):
                rb = jnp.full((16,), oh * _DIN, jnp.int32)
                for ch in range(12):
                    g0 = plsc.load_gather(hu, [rb + w0idx[ch]])
                    g1 = plsc.load_gather(hu, [rb + w1idx[ch]])
                    ob[slot, pl.ds(oh * _DOUT + ch * 16, 16)] = (
                        g0 + wfr[ch] * (g1 - g0)
                    )

            plsc.parallel_loop(0, _DOUT, unroll=1)(wbody)
            row = (c * _DOUT + od0) + k
            handles[slot] = pltpu.async_copy(
                ob.at[slot], o_hbm.at[row], sems.at[slot]
            )
    for h in handles:
        if h is not None:
            h.wait()


@jax.jit
def _sc_upsample(disp):
    a = jnp.reshape(disp, (_DIN, _ROW))
    mesh = plsc.VectorSubcoreMesh(core_axis_name="c", subcore_axis_name="s")
    out = pl.kernel(
        _sc_body,
        mesh=mesh,
        compiler_params=pltpu.CompilerParams(
            needs_layout_passes=False, use_tc_tiling_on_sc=False
        ),
        out_type=jax.ShapeDtypeStruct((3 * _DOUT, _OROW), jnp.float32),
        scratch_types=[
            pltpu.VMEM((_ROW,), jnp.float32),
            pltpu.VMEM((_ROW,), jnp.float32),
            pltpu.VMEM((_ROW,), jnp.float32),
            pltpu.VMEM((_DOUT * _DIN,), jnp.float32),
            pltpu.VMEM((2, _OROW), jnp.float32),
            pltpu.SemaphoreType.DMA((2,)),
        ],
    )(a)
    return jnp.reshape(out, (1, 3, _DOUT, _DOUT, _DOUT))


def kernel(kpts, disp, features_fixed):
    del kpts, features_fixed
    return _sc_upsample(disp)


# speedup vs baseline: 13.5574x; 6.3987x over previous
import functools

import jax
import jax.numpy as jnp
import numpy as np
from jax.experimental import pallas as pl

_DIN = 48
_DOUT = 192
_DT = 16
_NPLANES = _DT // 4 + 2


def _interp_matrix(n_in: int, n_out: int) -> np.ndarray:
    m = np.zeros((n_in, n_out), dtype=np.float32)
    scale = n_in / n_out
    for o in range(n_out):
        c = (o + 0.5) * scale - 0.5
        i0 = int(np.floor(c))
        t = c - i0
        m[min(max(i0, 0), n_in - 1), o] += 1.0 - t
        m[min(max(i0 + 1, 0), n_in - 1), o] += t
    return m


def _body(a_ref, mht_ref, mw_ref, o_ref):
    i = pl.program_id(0)
    mht = mht_ref[...]
    mw = mw_ref[...]
    d0 = i * (_DT // 4) - 1
    u = []
    for c in range(3):
        uc = []
        for j in range(_NPLANES):
            dj = jnp.clip(d0 + j, 0, _DIN - 1)
            s2 = jnp.dot(mht, a_ref[c, dj], preferred_element_type=jnp.float32)
            uc.append(jnp.dot(s2, mw, preferred_element_type=jnp.float32))
        u.append(uc)
    for k in range(_DT):
        i0rel = (2 * k - 3) // 8
        frac = k * 0.25 - 0.375 - i0rel
        j0 = i0rel + 1
        for c in range(3):
            o_ref[c, k] = (1.0 - frac) * u[c][j0] + frac * u[c][j0 + 1]


@jax.jit
def _upsample(disp):
    a = jnp.transpose(jnp.reshape(disp, (_DIN, _DIN, _DIN, 3)), (3, 0, 1, 2))
    mw = jnp.asarray(_interp_matrix(_DIN, _DOUT))
    mht = mw.T
    out = pl.pallas_call(
        _body,
        grid=(_DOUT // _DT,),
        in_specs=[
            pl.BlockSpec((3, _DIN, _DIN, _DIN), lambda i: (0, 0, 0, 0)),
            pl.BlockSpec((_DOUT, _DIN), lambda i: (0, 0)),
            pl.BlockSpec((_DIN, _DOUT), lambda i: (0, 0)),
        ],
        out_specs=pl.BlockSpec((3, _DT, _DOUT, _DOUT), lambda i: (0, i, 0, 0)),
        out_shape=jax.ShapeDtypeStruct((3, _DOUT, _DOUT, _DOUT), jnp.float32),
    )(a, mht, mw)
    return jnp.reshape(out, (1, 3, _DOUT, _DOUT, _DOUT))


def kernel(kpts, disp, features_fixed):
    del kpts, features_fixed
    return _upsample(disp)
